# baseline (speedup 1.0000x reference)
import functools

import jax
import jax.numpy as jnp
from jax import lax
from jax.experimental import pallas as pl
from jax.experimental.pallas import tpu as pltpu, tpu_sc as plsc

ROWS = 4096
COLS = 26
VOCAB = 1000
NUM_WORKERS = 32
PLANES_PER_WORKER = ROWS // NUM_WORKERS
L = 16


def _body(x_hbm, zeros_hbm, out_hbm, buf0, buf1, idx_v,
          sav00, sav01, sav10, sav11, sem0, sem1):
    wid = lax.axis_index("c") * 16 + lax.axis_index("s")
    base = wid * PLANES_PER_WORKER

    bufs = (buf0, buf1)
    sems = (sem0, sem1)
    saved = ((sav00, sav01), (sav10, sav11))

    pltpu.async_copy(zeros_hbm, buf0, sem0)
    pltpu.async_copy(zeros_hbm, buf1, sem1)

    pltpu.sync_copy(x_hbm.at[pl.ds(base, PLANES_PER_WORKER)], idx_v)

    iota = lax.iota(jnp.int32, L)
    rows0 = iota
    rows1 = jnp.minimum(iota + L, COLS - 1)
    mask1 = iota < (COLS - L)
    ones = jnp.full((L,), 1.0, jnp.float32)
    zeros_v = jnp.zeros((L,), jnp.float32)
    zeros_i = jnp.zeros((L,), jnp.int32)

    for pair in saved:
        for ref in pair:
            ref[...] = zeros_i

    def step(g, carry):
        for b in range(2):
            buf, sem, (sv0, sv1) = bufs[b], sems[b], saved[b]
            p = 2 * g + b
            pltpu.make_async_copy(zeros_hbm, buf, sem).wait()
            plsc.store_scatter(buf, [rows0, sv0[...]], zeros_v)
            plsc.store_scatter(buf, [rows1, sv1[...]], zeros_v, mask=mask1)
            p_vec = jnp.full((L,), p, jnp.int32)
            c0 = plsc.load_gather(idx_v, [p_vec, rows0])
            c1 = plsc.load_gather(idx_v, [p_vec, rows1], mask=mask1)
            c1 = jnp.where(mask1, c1, 0)
            plsc.store_scatter(buf, [rows0, c0], ones)
            plsc.store_scatter(buf, [rows1, c1], ones, mask=mask1)
            sv0[...] = c0
            sv1[...] = c1
            pltpu.async_copy(buf, out_hbm.at[base + p], sem)
        return carry

    lax.fori_loop(0, PLANES_PER_WORKER // 2, step, 0)

    pltpu.make_async_copy(zeros_hbm, buf0, sem0).wait()
    pltpu.make_async_copy(zeros_hbm, buf1, sem1).wait()


_onehot_sc = functools.partial(
    pl.kernel,
    out_type=jax.ShapeDtypeStruct((ROWS, COLS, VOCAB), jnp.float32),
    mesh=plsc.VectorSubcoreMesh(core_axis_name="c", subcore_axis_name="s"),
    compiler_params=pltpu.CompilerParams(
        use_tc_tiling_on_sc=False, needs_layout_passes=False),
    scratch_types=[
        pltpu.VMEM((COLS, VOCAB), jnp.float32),
        pltpu.VMEM((COLS, VOCAB), jnp.float32),
        pltpu.VMEM((PLANES_PER_WORKER, COLS), jnp.int32),
        pltpu.VMEM((L,), jnp.int32),
        pltpu.VMEM((L,), jnp.int32),
        pltpu.VMEM((L,), jnp.int32),
        pltpu.VMEM((L,), jnp.int32),
        pltpu.SemaphoreType.DMA,
        pltpu.SemaphoreType.DMA,
    ],
)(_body)


def kernel(x):
    zeros = jnp.zeros((COLS, VOCAB), jnp.float32)
    return _onehot_sc(x, zeros)
